# baseline (speedup 1.0000x reference)
import functools

import jax
import jax.numpy as jnp
from jax import lax
from jax.experimental import pallas as pl
from jax.experimental.pallas import tpu as pltpu
from jax.experimental.pallas import tpu_sc as plsc

N = 10000
E = 320000
G = 10
OUT = 16

NUM_CORES = 2
NUM_SUBCORES = 16
NUM_TILES = NUM_CORES * NUM_SUBCORES
EDGES_PER_TILE = E // NUM_TILES
CHUNK = 125
ITERS = EDGES_PER_TILE // CHUNK
ROWS_PER_SUBCORE = 624
TAIL_ROWS = N - NUM_SUBCORES * ROWS_PER_SUBCORE

_NBUF = {32: 10, 64: 8}


@functools.cache
def _sc_scatter(d):
    nbuf = _NBUF[d]
    steps = ITERS // nbuf
    mesh = plsc.VectorSubcoreMesh(core_axis_name="c", subcore_axis_name="s")

    @functools.partial(
        pl.kernel,
        out_type=jax.ShapeDtypeStruct((NUM_CORES, N, d), jnp.float32),
        mesh=mesh,
        scratch_types=[
            pltpu.VMEM((2, ITERS, CHUNK), jnp.int32),
            [pltpu.VMEM((CHUNK, d), jnp.float32)] * nbuf,
            [pltpu.SemaphoreType.DMA] * nbuf,
            [pltpu.SemaphoreType.DMA] * nbuf,
            pltpu.VMEM_SHARED((N, d), jnp.float32),
        ],
        compiler_params=pltpu.CompilerParams(use_tc_tiling_on_sc=False),
    )
    def sc_scatter(h_hbm, eidx_hbm, out_hbm,
                   idxv, bufs, gsems, ssems, acc):
        c = lax.axis_index("c")
        s = lax.axis_index("s")
        wid = c * NUM_SUBCORES + s
        rbase = s * ROWS_PER_SUBCORE
        pltpu.sync_copy(eidx_hbm.at[0, wid], idxv.at[0])
        pltpu.sync_copy(eidx_hbm.at[1, wid], idxv.at[1])
        zvec = jnp.zeros((16,), jnp.float32)

        def zrow(i, carry):
            for jj in range(d // 16):
                bufs[0][i, pl.ds(16 * jj, 16)] = zvec
            return carry

        lax.fori_loop(0, CHUNK, zrow, 0)
        for t in range(6):
            pltpu.sync_copy(bufs[0].at[pl.ds(0, 104)],
                            acc.at[pl.ds(rbase + t * 104, 104)])

        @pl.when(s == NUM_SUBCORES - 1)
        def _zero_tail():
            pltpu.sync_copy(
                bufs[0].at[pl.ds(0, TAIL_ROWS)],
                acc.at[pl.ds(NUM_SUBCORES * ROWS_PER_SUBCORE, TAIL_ROWS)])

        plsc.subcore_barrier()

        def body(j, carry):
            base = j * nbuf
            gds = [pltpu.async_copy(h_hbm.at[idxv.at[1, base + k]],
                                    bufs[k], gsems[k])
                   for k in range(nbuf)]
            sds = []
            for k in range(nbuf):
                gds[k].wait()
                sds.append(pltpu.async_copy(bufs[k],
                                            acc.at[idxv.at[0, base + k]],
                                            ssems[k], add=True))
            for k in range(nbuf):
                sds[k].wait()
            return carry

        lax.fori_loop(0, steps, body, 0)
        plsc.subcore_barrier()
        pltpu.sync_copy(acc.at[pl.ds(rbase, ROWS_PER_SUBCORE)],
                        out_hbm.at[c].at[pl.ds(rbase, ROWS_PER_SUBCORE)])

        @pl.when(s == NUM_SUBCORES - 1)
        def _write_tail():
            pltpu.sync_copy(
                acc.at[pl.ds(NUM_SUBCORES * ROWS_PER_SUBCORE, TAIL_ROWS)],
                out_hbm.at[c].at[pl.ds(NUM_SUBCORES * ROWS_PER_SUBCORE,
                                       TAIL_ROWS)])

    return sc_scatter


def _bdiag(wv, k):
    if k == 1:
        return wv
    din, do = wv.shape
    z = jnp.zeros((din, do), jnp.float32)
    cols = []
    for j in range(k):
        blocks = [wv if i == j else z for i in range(k)]
        cols.append(jnp.concatenate(blocks, axis=0))
    return jnp.concatenate(cols, axis=1)


def _project_body(x_ref, w_ref, z_ref):
    z_ref[...] = jnp.dot(x_ref[...], w_ref[...],
                         preferred_element_type=jnp.float32)


@functools.cache
def _tc_project(din, dout):
    return pl.pallas_call(
        _project_body,
        out_shape=jax.ShapeDtypeStruct((N, dout), jnp.float32))


@functools.cache
def _tc_layer(gf, dout, pre, out_mode, lidx):
    R = N // gf
    w = gf * dout

    def body(*refs):
        it = iter(refs)
        p_ref = next(it)
        hz_ref = next(it)
        gidt_ref = next(it)
        eps_ref = next(it)
        wbd_ref = None if pre else next(it)
        b_ref = next(it)
        gam_ref = next(it)
        bet_ref = next(it)
        pw_ref = next(it)
        pb_ref = next(it)
        wn_ref = next(it) if out_mode == "z" else None
        sin_ref = next(it)
        hout_ref = None if out_mode == "none" else next(it)
        sout_ref = next(it)

        eps = eps_ref[0, lidx]

        def tile_cols(row_ref):
            return jnp.concatenate([row_ref[...]] * gf, axis=1)

        base = p_ref[0] + p_ref[1] + (1.0 + eps) * hz_ref[...]
        bt = tile_cols(b_ref)
        if pre:
            rep = base + bt
        else:
            rep = jnp.dot(base, _bdiag(wbd_ref[...], gf),
                          preferred_element_type=jnp.float32) + bt
        def fold(row):
            acc = row[:, :dout]
            for j in range(1, gf):
                acc = acc + row[:, j * dout:(j + 1) * dout]
            return jnp.concatenate([acc] * gf, axis=1) if gf > 1 else acc

        mean_t = fold(jnp.sum(rep, axis=0, keepdims=True)) / N
        cent = rep - mean_t
        var_t = fold(jnp.sum(cent * cent, axis=0, keepdims=True)) / N
        hbn = (cent * lax.rsqrt(var_t + 1e-5) * tile_cols(gam_ref)
               + tile_cols(bet_ref))
        h = jnp.maximum(hbn, 0.0)
        if out_mode == "z":
            wn = _bdiag(wn_ref[...], w // wn_ref.shape[0])
            hout_ref[...] = jnp.dot(h, wn,
                                    preferred_element_type=jnp.float32)
        elif out_mode == "h":
            hout_ref[...] = h
        pg = jnp.zeros((G, dout), jnp.float32)
        for j in range(gf):
            onehot = (lax.broadcasted_iota(jnp.int32, (G, R), 0)
                      == gidt_ref[j][None, :]).astype(jnp.float32)
            pg = pg + jnp.dot(onehot, h[:, j * dout:(j + 1) * dout],
                              preferred_element_type=jnp.float32)
        sout_ref[...] = (sin_ref[...]
                         + jnp.dot(pg, pw_ref[...],
                                   preferred_element_type=jnp.float32)
                         + pb_ref[...])

    out_shape = []
    if out_mode != "none":
        out_shape.append(jax.ShapeDtypeStruct((R, w), jnp.float32))
    out_shape.append(jax.ShapeDtypeStruct((G, OUT), jnp.float32))
    return pl.pallas_call(body, out_shape=tuple(out_shape))


_SPECS = [
    (True, "z"),
    (True, "h"),
    (False, "z"),
    (True, "h"),
    (False, "none"),
]


def kernel(x, edge_index, graph_ids, eps, Ws, bs, gammas, betas, PWs, Pbs):
    eidx = edge_index.reshape(2, NUM_TILES, ITERS, CHUNK)
    gidt = {gf: graph_ids.reshape(N // gf, gf).T for gf in (2, 4)}
    eps2d = eps.reshape(1, -1)
    score = jnp.zeros((G, OUT), jnp.float32)
    hz = _tc_project(*Ws[0].shape)(x, Ws[0])
    for l, (pre, out_mode) in enumerate(_SPECS):
        dout = Ws[l].shape[1]
        width = Ws[l].shape[0] if not pre else dout
        gf = 128 // width
        p = _sc_scatter(width)(hz.reshape(N, width), eidx)
        args = [p.reshape(2, N // gf, 128), hz.reshape(N // gf, 128),
                gidt[gf], eps2d]
        if not pre:
            args.append(Ws[l])
        args += [bs[l].reshape(1, -1), gammas[l].reshape(1, -1),
                 betas[l].reshape(1, -1), PWs[l], Pbs[l].reshape(1, OUT)]
        if out_mode == "z":
            args.append(Ws[l + 1])
        args.append(score)
        outs = _tc_layer(gf, dout, pre, out_mode, l)(*args)
        if out_mode == "none":
            (score,) = outs
        else:
            hz, score = outs
    return score


# speedup vs baseline: 1.0088x; 1.0088x over previous
import functools

import jax
import jax.numpy as jnp
from jax import lax
from jax.experimental import pallas as pl
from jax.experimental.pallas import tpu as pltpu
from jax.experimental.pallas import tpu_sc as plsc

N = 10000
E = 320000
G = 10
OUT = 16

NUM_CORES = 2
NUM_SUBCORES = 16
NUM_TILES = NUM_CORES * NUM_SUBCORES
EDGES_PER_TILE = E // NUM_TILES
CHUNK = 125
ITERS = EDGES_PER_TILE // CHUNK
ROWS_PER_SUBCORE = 624
TAIL_ROWS = N - NUM_SUBCORES * ROWS_PER_SUBCORE

_NBUF = {32: 10, 64: 8}


@functools.cache
def _sc_scatter(d):
    nbuf = _NBUF[d]
    steps = ITERS // nbuf
    mesh = plsc.VectorSubcoreMesh(core_axis_name="c", subcore_axis_name="s")

    @functools.partial(
        pl.kernel,
        out_type=jax.ShapeDtypeStruct((NUM_CORES, N, d), jnp.float32),
        mesh=mesh,
        scratch_types=[
            pltpu.VMEM((ITERS, 2, CHUNK), jnp.int32),
            [pltpu.VMEM((CHUNK, d), jnp.float32)] * nbuf,
            [pltpu.SemaphoreType.DMA] * nbuf,
            [pltpu.SemaphoreType.DMA] * nbuf,
            pltpu.VMEM_SHARED((N, d), jnp.float32),
        ],
        compiler_params=pltpu.CompilerParams(use_tc_tiling_on_sc=False),
    )
    def sc_scatter(h_hbm, eidx_hbm, out_hbm,
                   idxv, bufs, gsems, ssems, acc):
        c = lax.axis_index("c")
        s = lax.axis_index("s")
        wid = c * NUM_SUBCORES + s
        rbase = s * ROWS_PER_SUBCORE
        pltpu.sync_copy(eidx_hbm.at[wid], idxv)
        zvec = jnp.zeros((16,), jnp.float32)

        def zrow(i, carry):
            for jj in range(d // 16):
                bufs[0][i, pl.ds(16 * jj, 16)] = zvec
            return carry

        lax.fori_loop(0, CHUNK, zrow, 0)
        for t in range(6):
            pltpu.sync_copy(bufs[0].at[pl.ds(0, 104)],
                            acc.at[pl.ds(rbase + t * 104, 104)])

        @pl.when(s == NUM_SUBCORES - 1)
        def _zero_tail():
            pltpu.sync_copy(
                bufs[0].at[pl.ds(0, TAIL_ROWS)],
                acc.at[pl.ds(NUM_SUBCORES * ROWS_PER_SUBCORE, TAIL_ROWS)])

        plsc.subcore_barrier()

        def body(j, carry):
            base = j * nbuf
            gds = [pltpu.async_copy(h_hbm.at[idxv.at[base + k, 1]],
                                    bufs[k], gsems[k])
                   for k in range(nbuf)]
            sds = []
            for k in range(nbuf):
                gds[k].wait()
                sds.append(pltpu.async_copy(bufs[k],
                                            acc.at[idxv.at[base + k, 0]],
                                            ssems[k], add=True))
            for k in range(nbuf):
                sds[k].wait()
            return carry

        lax.fori_loop(0, steps, body, 0)
        plsc.subcore_barrier()
        pltpu.sync_copy(acc.at[pl.ds(rbase, ROWS_PER_SUBCORE)],
                        out_hbm.at[c].at[pl.ds(rbase, ROWS_PER_SUBCORE)])

        @pl.when(s == NUM_SUBCORES - 1)
        def _write_tail():
            pltpu.sync_copy(
                acc.at[pl.ds(NUM_SUBCORES * ROWS_PER_SUBCORE, TAIL_ROWS)],
                out_hbm.at[c].at[pl.ds(NUM_SUBCORES * ROWS_PER_SUBCORE,
                                       TAIL_ROWS)])

    return sc_scatter


def _bdiag(wv, k):
    if k == 1:
        return wv
    din, do = wv.shape
    z = jnp.zeros((din, do), jnp.float32)
    cols = []
    for j in range(k):
        blocks = [wv if i == j else z for i in range(k)]
        cols.append(jnp.concatenate(blocks, axis=0))
    return jnp.concatenate(cols, axis=1)


def _project_body(x_ref, w_ref, z_ref):
    z_ref[...] = jnp.dot(x_ref[...], _bdiag(w_ref[...], 4),
                         preferred_element_type=jnp.float32)


@functools.cache
def _tc_project(rows):
    return pl.pallas_call(
        _project_body,
        out_shape=jax.ShapeDtypeStruct((rows, 128), jnp.float32))


@functools.cache
def _tc_layer(gf, dout, pre, out_mode, lidx):
    R = N // gf
    w = gf * dout

    def body(*refs):
        it = iter(refs)
        p_ref = next(it)
        hz_ref = next(it)
        gidt_ref = next(it)
        eps_ref = next(it)
        wbd_ref = None if pre else next(it)
        b_ref = next(it)
        gam_ref = next(it)
        bet_ref = next(it)
        pw_ref = next(it)
        pb_ref = next(it)
        wn_ref = next(it) if out_mode == "z" else None
        sin_ref = next(it)
        hout_ref = None if out_mode == "none" else next(it)
        sout_ref = next(it)

        eps = eps_ref[0, lidx]

        def tile_cols(row_ref):
            return jnp.concatenate([row_ref[...]] * gf, axis=1)

        base = p_ref[0] + p_ref[1] + (1.0 + eps) * hz_ref[...]
        bt = tile_cols(b_ref)
        if pre:
            rep = base + bt
        else:
            rep = jnp.dot(base, _bdiag(wbd_ref[...], gf),
                          preferred_element_type=jnp.float32) + bt
        def fold(row):
            acc = row[:, :dout]
            for j in range(1, gf):
                acc = acc + row[:, j * dout:(j + 1) * dout]
            return jnp.concatenate([acc] * gf, axis=1) if gf > 1 else acc

        mean_t = fold(jnp.sum(rep, axis=0, keepdims=True)) / N
        cent = rep - mean_t
        var_t = fold(jnp.sum(cent * cent, axis=0, keepdims=True)) / N
        hbn = (cent * lax.rsqrt(var_t + 1e-5) * tile_cols(gam_ref)
               + tile_cols(bet_ref))
        h = jnp.maximum(hbn, 0.0)
        if out_mode == "z":
            wn = _bdiag(wn_ref[...], w // wn_ref.shape[0])
            hout_ref[...] = jnp.dot(h, wn,
                                    preferred_element_type=jnp.float32)
        elif out_mode == "h":
            hout_ref[...] = h
        pg = jnp.zeros((G, dout), jnp.float32)
        for j in range(gf):
            onehot = (lax.broadcasted_iota(jnp.int32, (G, R), 0)
                      == gidt_ref[j][None, :]).astype(jnp.float32)
            pg = pg + jnp.dot(onehot, h[:, j * dout:(j + 1) * dout],
                              preferred_element_type=jnp.float32)
        sout_ref[...] = (sin_ref[...]
                         + jnp.dot(pg, pw_ref[...],
                                   preferred_element_type=jnp.float32)
                         + pb_ref[...])

    out_shape = []
    if out_mode != "none":
        out_shape.append(jax.ShapeDtypeStruct((R, w), jnp.float32))
    out_shape.append(jax.ShapeDtypeStruct((G, OUT), jnp.float32))
    return pl.pallas_call(body, out_shape=tuple(out_shape))


_SPECS = [
    (True, "z"),
    (True, "h"),
    (False, "z"),
    (True, "h"),
    (False, "none"),
]


def kernel(x, edge_index, graph_ids, eps, Ws, bs, gammas, betas, PWs, Pbs):
    eidx = edge_index.reshape(2, NUM_TILES, ITERS, CHUNK).transpose(1, 2, 0, 3)
    gidt = {gf: graph_ids.reshape(N // gf, gf).T for gf in (2, 4)}
    eps2d = eps.reshape(1, -1)
    score = jnp.zeros((G, OUT), jnp.float32)
    hz = _tc_project(N // 4)(x.reshape(N // 4, 512), Ws[0])
    for l, (pre, out_mode) in enumerate(_SPECS):
        dout = Ws[l].shape[1]
        width = Ws[l].shape[0] if not pre else dout
        gf = 128 // width
        p = _sc_scatter(width)(hz.reshape(N, width), eidx)
        args = [p.reshape(2, N // gf, 128), hz.reshape(N // gf, 128),
                gidt[gf], eps2d]
        if not pre:
            args.append(Ws[l])
        args += [bs[l].reshape(1, -1), gammas[l].reshape(1, -1),
                 betas[l].reshape(1, -1), PWs[l], Pbs[l].reshape(1, OUT)]
        if out_mode == "z":
            args.append(Ws[l + 1])
        args.append(score)
        outs = _tc_layer(gf, dout, pre, out_mode, l)(*args)
        if out_mode == "none":
            (score,) = outs
        else:
            hz, score = outs
    return score
